# baseline (speedup 1.0000x reference)
import functools

import jax
import jax.numpy as jnp
from jax import lax
from jax.experimental import pallas as pl
from jax.experimental.pallas import tpu as pltpu
from jax.experimental.pallas import tpu_sc as plsc

_T0 = 128
_NT = 2


def _emb_lookup(table, ids2, B, L, D, NC, NW, n_ch, LP, K):
    mesh = plsc.VectorSubcoreMesh(core_axis_name="c", subcore_axis_name="s")
    tail0 = _NT * _T0

    @functools.partial(
        pl.kernel,
        mesh=mesh,
        out_type=jax.ShapeDtypeStruct((B, L, D), table.dtype),
        compiler_params=pltpu.CompilerParams(needs_layout_passes=False),
        scratch_types=(
            [pltpu.VMEM((n_ch * LP,), jnp.int32)]
            + [pltpu.VMEM((L, D), table.dtype) for _ in range(K)]
            + [pltpu.VMEM((L, _T0), table.dtype) for _ in range(K)]
            + [pltpu.SemaphoreType.DMA for _ in range(4 * K)]
        ),
    )
    def emb(table_hbm, ids_hbm, out_hbm, idx_v, *rest):
        bufs = rest[:K]
        tbufs = rest[K : 2 * K]
        gsemA = rest[2 * K : 3 * K]
        gsemB = rest[3 * K : 4 * K]
        gsemC = rest[4 * K : 5 * K]
        osem = rest[5 * K : 6 * K]
        wid = lax.axis_index("s") * NC + lax.axis_index("c")
        base = wid * n_ch
        pltpu.sync_copy(ids_hbm.at[wid], idx_v)

        def start_gathers(c, b):
            idx = idx_v.at[pl.ds(c * LP, L)]
            pltpu.async_copy(
                table_hbm.at[:, pl.ds(0, _T0)].at[idx],
                bufs[b].at[:, pl.ds(0, _T0)],
                gsemA[b],
            )
            pltpu.async_copy(
                table_hbm.at[:, pl.ds(_T0, _T0)].at[idx],
                bufs[b].at[:, pl.ds(_T0, _T0)],
                gsemB[b],
            )
            pltpu.async_copy(
                table_hbm.at[:, pl.ds(tail0, _T0)].at[idx], tbufs[b], gsemC[b]
            )

        def wait_gathers(c, b):
            idx = idx_v.at[pl.ds(c * LP, L)]
            pltpu.make_async_copy(
                table_hbm.at[:, pl.ds(0, _T0)].at[idx],
                bufs[b].at[:, pl.ds(0, _T0)],
                gsemA[b],
            ).wait()
            pltpu.make_async_copy(
                table_hbm.at[:, pl.ds(_T0, _T0)].at[idx],
                bufs[b].at[:, pl.ds(_T0, _T0)],
                gsemB[b],
            ).wait()
            pltpu.make_async_copy(
                table_hbm.at[:, pl.ds(tail0, _T0)].at[idx], tbufs[b], gsemC[b]
            ).wait()

        for b in range(K - 1):
            start_gathers(b, b)

        @pl.loop(0, n_ch // K)
        def _outer(g):
            c0 = g * K
            for b in range(K):
                c = c0 + b
                wait_gathers(c, b)

                @pl.loop(0, L)
                def _row(r):
                    buf = bufs[b]
                    tb = tbufs[b]
                    buf[r, pl.ds(tail0, 16)] = tb[r, pl.ds(0, 16)]
                    buf[r, pl.ds(tail0 + 16, 16)] = tb[r, pl.ds(16, 16)]
                    v = tb[r, pl.ds(32, 16)]
                    cols = lax.iota(jnp.int32, 16) + (tail0 + 32)
                    rows = jnp.full((16,), r, jnp.int32)
                    plsc.store_scatter(
                        bufs[b], [rows, cols], v, mask=cols < D
                    )

                pltpu.async_copy(bufs[b], out_hbm.at[base + c], osem[b])
                nb = (b + K - 1) % K
                nxt = c + K - 1

                @pl.when(nxt < n_ch)
                def _():
                    @pl.when(c >= 1)
                    def _():
                        pltpu.make_async_copy(
                            bufs[nb], out_hbm.at[base + c - 1], osem[nb]
                        ).wait()

                    start_gathers(nxt, nb)

        for b in range(K):
            pltpu.make_async_copy(
                bufs[b], out_hbm.at[base + n_ch - K + b], osem[b]
            ).wait()

    return emb(table, ids2)


def kernel(table, _input_token_ids):
    V, D = table.shape
    Bt, Lt = _input_token_ids.shape
    info = plsc.get_sparse_core_info()
    NC = info.num_cores
    NW = NC * info.num_subcores
    K = 4
    LP = 56
    assert Bt % NW == 0
    n_ch = Bt // NW
    assert n_ch % K == 0
    tpad = jnp.pad(table, ((0, 0), (0, (_NT + 1) * _T0 - D)))
    ids2 = jnp.pad(_input_token_ids, ((0, 0), (0, LP - Lt))).reshape(
        NW, n_ch * LP
    )
    return _emb_lookup(tpad, ids2, Bt, Lt, D, NC, NW, n_ch, LP, K)


# speedup vs baseline: 1.2812x; 1.2812x over previous
import functools

import jax
import jax.numpy as jnp
from jax import lax
from jax.experimental import pallas as pl
from jax.experimental.pallas import tpu as pltpu
from jax.experimental.pallas import tpu_sc as plsc

_T0 = 128
_NT = 2


def _emb_lookup(table, t3, ids2, B, L, D, NC, NW, n_ch, LP, K):
    mesh = plsc.VectorSubcoreMesh(core_axis_name="c", subcore_axis_name="s")
    tail0 = _NT * _T0

    @functools.partial(
        pl.kernel,
        mesh=mesh,
        out_type=jax.ShapeDtypeStruct((B, L, D), table.dtype),
        compiler_params=pltpu.CompilerParams(needs_layout_passes=False),
        scratch_types=(
            [pltpu.VMEM((n_ch * LP,), jnp.int32)]
            + [pltpu.VMEM((L, D), table.dtype) for _ in range(K)]
            + [pltpu.VMEM((L, _T0), table.dtype) for _ in range(K)]
            + [pltpu.SemaphoreType.DMA for _ in range(4 * K)]
        ),
    )
    def emb(table_hbm, t3_hbm, ids_hbm, out_hbm, idx_v, *rest):
        bufs = rest[:K]
        tbufs = rest[K : 2 * K]
        gsemA = rest[2 * K : 3 * K]
        gsemB = rest[3 * K : 4 * K]
        gsemC = rest[4 * K : 5 * K]
        osem = rest[5 * K : 6 * K]
        wid = lax.axis_index("s") * NC + lax.axis_index("c")
        base = wid * n_ch
        pltpu.sync_copy(ids_hbm.at[wid], idx_v)

        def start_gathers(c, b):
            idx = idx_v.at[pl.ds(c * LP, L)]
            pltpu.async_copy(
                table_hbm.at[:, pl.ds(0, _T0)].at[idx],
                bufs[b].at[:, pl.ds(0, _T0)],
                gsemA[b],
            )
            pltpu.async_copy(
                table_hbm.at[:, pl.ds(_T0, _T0)].at[idx],
                bufs[b].at[:, pl.ds(_T0, _T0)],
                gsemB[b],
            )
            pltpu.async_copy(t3_hbm.at[idx], tbufs[b], gsemC[b])

        def wait_gathers(c, b):
            idx = idx_v.at[pl.ds(c * LP, L)]
            pltpu.make_async_copy(
                table_hbm.at[:, pl.ds(0, _T0)].at[idx],
                bufs[b].at[:, pl.ds(0, _T0)],
                gsemA[b],
            ).wait()
            pltpu.make_async_copy(
                table_hbm.at[:, pl.ds(_T0, _T0)].at[idx],
                bufs[b].at[:, pl.ds(_T0, _T0)],
                gsemB[b],
            ).wait()
            pltpu.make_async_copy(t3_hbm.at[idx], tbufs[b], gsemC[b]).wait()

        for b in range(K - 1):
            start_gathers(b, b)

        @pl.loop(0, n_ch // K)
        def _outer(g):
            c0 = g * K
            for b in range(K):
                c = c0 + b
                wait_gathers(c, b)

                @pl.loop(0, L)
                def _row(r):
                    buf = bufs[b]
                    tb = tbufs[b]
                    buf[r, pl.ds(tail0, 16)] = tb[r, pl.ds(0, 16)]
                    buf[r, pl.ds(tail0 + 16, 16)] = tb[r, pl.ds(16, 16)]
                    v = tb[r, pl.ds(32, 16)]
                    cols = lax.iota(jnp.int32, 16) + (tail0 + 32)
                    rows = jnp.full((16,), r, jnp.int32)
                    plsc.store_scatter(
                        bufs[b], [rows, cols], v, mask=cols < D
                    )

                pltpu.async_copy(bufs[b], out_hbm.at[base + c], osem[b])
                nb = (b + K - 1) % K
                nxt = c + K - 1

                @pl.when(nxt < n_ch)
                def _():
                    @pl.when(c >= 1)
                    def _():
                        pltpu.make_async_copy(
                            bufs[nb], out_hbm.at[base + c - 1], osem[nb]
                        ).wait()

                    start_gathers(nxt, nb)

        for b in range(K):
            pltpu.make_async_copy(
                bufs[b], out_hbm.at[base + n_ch - K + b], osem[b]
            ).wait()

    return emb(table, t3, ids2)


def kernel(table, _input_token_ids):
    V, D = table.shape
    Bt, Lt = _input_token_ids.shape
    info = plsc.get_sparse_core_info()
    NC = info.num_cores
    NW = NC * info.num_subcores
    K = 4
    LP = 56
    assert Bt % NW == 0
    n_ch = Bt // NW
    assert n_ch % K == 0
    t3 = jnp.pad(table[:, _NT * _T0 :], ((0, 0), (0, (_NT + 1) * _T0 - D)))
    idsp = jnp.pad(_input_token_ids, ((0, 0), (0, LP - Lt)))
    halves = []
    Bh = Bt // 2
    for h in range(2):
        ids2 = idsp[h * Bh : (h + 1) * Bh].reshape(NW, (n_ch // 2) * LP)
        halves.append(
            _emb_lookup(
                table, t3, ids2, Bh, Lt, D, NC, NW, n_ch // 2, LP, K
            )
        )
    return jnp.concatenate(halves, axis=0)


# speedup vs baseline: 1.5405x; 1.2024x over previous
import functools

import jax
import jax.numpy as jnp
from jax import lax
from jax.experimental import pallas as pl
from jax.experimental.pallas import tpu as pltpu
from jax.experimental.pallas import tpu_sc as plsc

_T0 = 128
_NT = 2


def _emb_lookup(table, t3, ids2, B, L, D, NC, NW, n_ch, LP, K):
    mesh = plsc.VectorSubcoreMesh(core_axis_name="c", subcore_axis_name="s")
    tail0 = _NT * _T0

    @functools.partial(
        pl.kernel,
        mesh=mesh,
        out_type=jax.ShapeDtypeStruct((B, L, D), table.dtype),
        compiler_params=pltpu.CompilerParams(needs_layout_passes=False),
        scratch_types=(
            [pltpu.VMEM((n_ch * LP,), jnp.int32)]
            + [pltpu.VMEM((L, D), table.dtype) for _ in range(K)]
            + [pltpu.VMEM((L, _T0), table.dtype) for _ in range(K)]
            + [pltpu.SemaphoreType.DMA for _ in range(4 * K)]
        ),
    )
    def emb(table_hbm, t3_hbm, ids_hbm, out_hbm, idx_v, *rest):
        bufs = rest[:K]
        tbufs = rest[K : 2 * K]
        gsemA = rest[2 * K : 3 * K]
        gsemB = rest[3 * K : 4 * K]
        gsemC = rest[4 * K : 5 * K]
        osem = rest[5 * K : 6 * K]
        wid = lax.axis_index("s") * NC + lax.axis_index("c")
        base = wid * n_ch
        pltpu.sync_copy(ids_hbm.at[wid], idx_v)

        def start_gathers(c, b):
            idx = idx_v.at[pl.ds(c * LP, L)]
            pltpu.async_copy(
                table_hbm.at[:, pl.ds(0, _T0)].at[idx],
                bufs[b].at[:, pl.ds(0, _T0)],
                gsemA[b],
            )
            pltpu.async_copy(
                table_hbm.at[:, pl.ds(_T0, _T0)].at[idx],
                bufs[b].at[:, pl.ds(_T0, _T0)],
                gsemB[b],
            )
            pltpu.async_copy(t3_hbm.at[idx], tbufs[b], gsemC[b])

        def wait_gathers(c, b):
            idx = idx_v.at[pl.ds(c * LP, L)]
            pltpu.make_async_copy(
                table_hbm.at[:, pl.ds(0, _T0)].at[idx],
                bufs[b].at[:, pl.ds(0, _T0)],
                gsemA[b],
            ).wait()
            pltpu.make_async_copy(
                table_hbm.at[:, pl.ds(_T0, _T0)].at[idx],
                bufs[b].at[:, pl.ds(_T0, _T0)],
                gsemB[b],
            ).wait()
            pltpu.make_async_copy(t3_hbm.at[idx], tbufs[b], gsemC[b]).wait()

        for b in range(K - 1):
            start_gathers(b, b)

        @pl.loop(0, n_ch // K)
        def _outer(g):
            c0 = g * K
            for b in range(K):
                c = c0 + b
                wait_gathers(c, b)

                @pl.loop(0, L)
                def _row(r):
                    buf = bufs[b]
                    tb = tbufs[b]
                    buf[r, pl.ds(tail0, 16)] = tb[r, pl.ds(0, 16)]
                    buf[r, pl.ds(tail0 + 16, 16)] = tb[r, pl.ds(16, 16)]
                    v = tb[r, pl.ds(32, 16)]
                    cols = lax.iota(jnp.int32, 16) + (tail0 + 32)
                    rows = jnp.full((16,), r, jnp.int32)
                    plsc.store_scatter(
                        bufs[b], [rows, cols], v, mask=cols < D
                    )

                pltpu.async_copy(bufs[b], out_hbm.at[base + c], osem[b])
                nb = (b + K - 1) % K
                nxt = c + K - 1

                @pl.when(nxt < n_ch)
                def _():
                    @pl.when(c >= 1)
                    def _():
                        pltpu.make_async_copy(
                            bufs[nb], out_hbm.at[base + c - 1], osem[nb]
                        ).wait()

                    start_gathers(nxt, nb)

        for b in range(K):
            pltpu.make_async_copy(
                bufs[b], out_hbm.at[base + n_ch - K + b], osem[b]
            ).wait()

    return emb(table, t3, ids2)


def kernel(table, _input_token_ids):
    V, D = table.shape
    Bt, Lt = _input_token_ids.shape
    info = plsc.get_sparse_core_info()
    NC = info.num_cores
    NW = NC * info.num_subcores
    K = 4
    LP = 56
    assert Bt % NW == 0
    n_ch = Bt // NW
    assert n_ch % K == 0
    t3 = jnp.concatenate(
        [
            table[:, _NT * _T0 :],
            jnp.zeros((V, (_NT + 1) * _T0 - D), table.dtype),
        ],
        axis=1,
    )
    ids2 = jnp.pad(_input_token_ids, ((0, 0), (0, LP - Lt))).reshape(
        NW, n_ch * LP
    )
    return _emb_lookup(table, t3, ids2, Bt, Lt, D, NC, NW, n_ch, LP, K)
